# baseline (speedup 1.0000x reference)
import functools

import jax
import jax.numpy as jnp
from jax import lax
from jax.experimental import pallas as pl
from jax.experimental.pallas import tpu as pltpu
from jax.experimental.pallas import tpu_sc as plsc

_VOCAB = 1000000
_D = 64
_B = 4096
_L = 200
_NC = 2
_NS = 16
_NW = _NC * _NS
_LANES = 16
_SCALE = 8.0

_VBLK = 128
_NVB = -(-_VOCAB // _VBLK)
_A_ITERS = -(-_NVB // _NW)
_A_NBUF = 4
_VLAST = _VOCAB - (_NVB - 1) * _VBLK

_BBLK = _B // _NW
_B_NBUF = 4
_B_NGROUP = _L // _B_NBUF


def _prep_body(tt_hbm, tp_hbm, *rest):
    ibufs = rest[0:_A_NBUF]
    obufs = rest[_A_NBUF:2 * _A_NBUF]
    isems = rest[2 * _A_NBUF:3 * _A_NBUF]
    osems = rest[3 * _A_NBUF:4 * _A_NBUF]

    wid = lax.axis_index("s") * _NC + lax.axis_index("c")
    iota = lax.iota(jnp.int32, _LANES)

    def blk(it):
        return wid + it * _NW

    def in_copy(it, b):
        return pltpu.make_async_copy(
            tt_hbm.at[:, pl.ds(blk(it) * _VBLK, _VBLK)], ibufs[b], isems[b])

    def out_copy(it, b):
        return pltpu.make_async_copy(
            obufs[b], tp_hbm.at[pl.ds(blk(it) * (_VBLK // 2), _VBLK // 2)],
            osems[b])

    def out_copy_last(it, b):
        return pltpu.make_async_copy(
            obufs[b].at[pl.ds(0, _VLAST // 2)],
            tp_hbm.at[pl.ds(blk(it) * (_VBLK // 2), _VLAST // 2)],
            osems[b])

    for b in range(_A_NBUF):
        @pl.when(blk(b) < _NVB)
        def _(b=b):
            in_copy(b, b).start()

    def step(it, b):
        bi = blk(it)

        @pl.when(bi < _NVB)
        def _():
            in_copy(it, b).wait()

            @pl.when(it >= _A_NBUF)
            def _():
                @pl.when(blk(it - _A_NBUF) < _NVB - 1)
                def _():
                    out_copy(it - _A_NBUF, b).wait()

                @pl.when(blk(it - _A_NBUF) == _NVB - 1)
                def _():
                    out_copy_last(it - _A_NBUF, b).wait()

            ibuf = ibufs[b]
            obuf = obufs[b]

            def row(k, acc, ibuf=ibuf, obuf=obuf):
                for h in range(2):
                    vi = jnp.full((_LANES,), 0, jnp.int32) + (2 * k + h)
                    for kk in range(_D // _LANES):
                        di = iota + (kk * _LANES)
                        vals = plsc.load_gather(ibuf, [di, vi])
                        obuf[k, pl.ds(h * _D + kk * _LANES, _LANES)] = (
                            vals * _SCALE)
                return acc

            lax.fori_loop(0, _VBLK // 2, row, 0, unroll=4)

            @pl.when(bi < _NVB - 1)
            def _():
                out_copy(it, b).start()

            @pl.when(bi == _NVB - 1)
            def _():
                out_copy_last(it, b).start()

            @pl.when(blk(it + _A_NBUF) < _NVB)
            def _():
                in_copy(it + _A_NBUF, b).start()

    def group(g, carry):
        for b in range(_A_NBUF):
            step(g * _A_NBUF + b, b)
        return carry

    lax.fori_loop(0, -(-_A_ITERS // _A_NBUF), group, 0)

    for b in range(_A_NBUF):
        last_groups = -(-_A_ITERS // _A_NBUF)
        it = (last_groups - 1) * _A_NBUF + b
        for back in range(2):
            itb = it - back * _A_NBUF

            @pl.when((blk(itb) >= _NVB - 1 - (_NW - 1)) &
                     (blk(itb) < _NVB - 1))
            def _(itb=itb, b=b):
                out_copy(itb, b).wait()

            @pl.when(blk(itb) == _NVB - 1)
            def _(itb=itb, b=b):
                out_copy_last(itb, b).wait()


_prep_call = functools.partial(
    pl.kernel,
    out_type=jax.ShapeDtypeStruct((_VOCAB // 2, 2 * _D), jnp.float32),
    mesh=plsc.VectorSubcoreMesh(core_axis_name="c", subcore_axis_name="s"),
    compiler_params=pltpu.CompilerParams(needs_layout_passes=False),
    scratch_types=(
        [pltpu.VMEM((_D, _VBLK), jnp.float32) for _ in range(_A_NBUF)]
        + [pltpu.VMEM((_VBLK // 2, 2 * _D), jnp.float32)
           for _ in range(_A_NBUF)]
        + [pltpu.SemaphoreType.DMA for _ in range(2 * _A_NBUF)]
    ),
)(_prep_body)


def _lookup_body(tp_hbm, xt_hbm, out_hbm, idx_v, *rest):
    pidx = rest[0:_B_NBUF]
    lohi = rest[_B_NBUF:2 * _B_NBUF]
    gbufs = rest[2 * _B_NBUF:3 * _B_NBUF]
    obufs = rest[3 * _B_NBUF:4 * _B_NBUF]
    gsems = rest[4 * _B_NBUF:5 * _B_NBUF]
    osems = rest[5 * _B_NBUF:6 * _B_NBUF]

    wid = lax.axis_index("s") * _NC + lax.axis_index("c")
    col0 = wid * _BBLK
    iota = lax.iota(jnp.int32, _LANES)

    pltpu.sync_copy(xt_hbm.at[:, pl.ds(col0, _BBLK)], idx_v)

    def build_idx(c, b):
        for k in range(_BBLK // _LANES):
            sl = pl.ds(k * _LANES, _LANES)
            v = idx_v[c, sl]
            pidx[b][sl] = lax.shift_right_logical(v, 1)
            lohi[b][sl] = lax.bitwise_and(v, 1) * _D

    def gather_copy(b):
        return pltpu.make_async_copy(tp_hbm.at[pidx[b]], gbufs[b], gsems[b])

    def out_copy(c, b):
        return pltpu.make_async_copy(
            obufs[b], out_hbm.at[c, :, pl.ds(col0, _BBLK)], osems[b])

    for b in range(_B_NBUF):
        build_idx(b, b)
        gather_copy(b).start()

    def group(g, carry):
        for b in range(_B_NBUF):
            c = g * _B_NBUF + b
            gather_copy(b).wait()

            @pl.when(g > 0)
            def _():
                out_copy(c - _B_NBUF, b).wait()

            gbuf = gbufs[b]
            obuf = obufs[b]
            lob = lohi[b]

            for k in range(_BBLK // _LANES):
                ji = iota + (k * _LANES)
                hi = lob[pl.ds(k * _LANES, _LANES)]

                def drow(d, acc, ji=ji, hi=hi, k=k, gbuf=gbuf, obuf=obuf):
                    vals = plsc.load_gather(gbuf, [ji, hi + d])
                    obuf[d, pl.ds(k * _LANES, _LANES)] = vals
                    return acc

                lax.fori_loop(0, _D, drow, 0, unroll=8)

            out_copy(c, b).start()

            @pl.when(g + 1 < _B_NGROUP)
            def _():
                build_idx(c + _B_NBUF, b)
                gather_copy(b).start()
        return carry

    lax.fori_loop(0, _B_NGROUP, group, 0)

    for b in range(_B_NBUF):
        out_copy((_B_NGROUP - 1) * _B_NBUF + b, b).wait()


_lookup_call = functools.partial(
    pl.kernel,
    out_type=jax.ShapeDtypeStruct((_L, _D, _B), jnp.float32),
    mesh=plsc.VectorSubcoreMesh(core_axis_name="c", subcore_axis_name="s"),
    compiler_params=pltpu.CompilerParams(needs_layout_passes=False),
    scratch_types=(
        [pltpu.VMEM((_L, _BBLK), jnp.int32)]
        + [pltpu.VMEM((_BBLK,), jnp.int32) for _ in range(2 * _B_NBUF)]
        + [pltpu.VMEM((_BBLK, 2 * _D), jnp.float32) for _ in range(_B_NBUF)]
        + [pltpu.VMEM((_D, _BBLK), jnp.float32) for _ in range(_B_NBUF)]
        + [pltpu.SemaphoreType.DMA for _ in range(2 * _B_NBUF)]
    ),
)(_lookup_body)


@jax.jit
def kernel(embedding_table, x):
    tp = _prep_call(embedding_table.T)
    out = _lookup_call(tp, x.T)
    return out.transpose(2, 0, 1)


# speedup vs baseline: 1.9843x; 1.9843x over previous
import functools

import jax
import jax.numpy as jnp
from jax import lax
from jax.experimental import pallas as pl
from jax.experimental.pallas import tpu as pltpu
from jax.experimental.pallas import tpu_sc as plsc

_VOCAB = 1000000
_D = 64
_B = 4096
_L = 200
_NC = 2
_NS = 16
_NW = _NC * _NS
_LANES = 16
_SCALE = 8.0

_VBLK = 128
_NVB = -(-_VOCAB // _VBLK)
_A_ITERS = -(-_NVB // _NW)
_A_NBUF = 4
_VLAST = _VOCAB - (_NVB - 1) * _VBLK

_BBLK = _B // _NW
_B_NBUF = 4
_B_NGROUP = _L // _B_NBUF


def _prep_body(tt_hbm, tp_hbm, *rest):
    ibufs = rest[0:_A_NBUF]
    obufs = rest[_A_NBUF:2 * _A_NBUF]
    isems = rest[2 * _A_NBUF:3 * _A_NBUF]
    osems = rest[3 * _A_NBUF:4 * _A_NBUF]

    wid = lax.axis_index("s") * _NC + lax.axis_index("c")
    iota = lax.iota(jnp.int32, _LANES)

    def blk(it):
        return wid + it * _NW

    def in_copy(it, b):
        return pltpu.make_async_copy(
            tt_hbm.at[:, pl.ds(blk(it) * _VBLK, _VBLK)], ibufs[b], isems[b])

    def out_copy(it, b):
        return pltpu.make_async_copy(
            obufs[b], tp_hbm.at[pl.ds(blk(it) * (_VBLK // 2), _VBLK // 2)],
            osems[b])

    def out_copy_last(it, b):
        return pltpu.make_async_copy(
            obufs[b].at[pl.ds(0, _VLAST // 2)],
            tp_hbm.at[pl.ds(blk(it) * (_VBLK // 2), _VLAST // 2)],
            osems[b])

    for b in range(_A_NBUF):
        @pl.when(blk(b) < _NVB)
        def _(b=b):
            in_copy(b, b).start()

    def step(it, b):
        bi = blk(it)

        @pl.when(bi < _NVB)
        def _():
            in_copy(it, b).wait()

            @pl.when(it >= _A_NBUF)
            def _():
                @pl.when(blk(it - _A_NBUF) < _NVB - 1)
                def _():
                    out_copy(it - _A_NBUF, b).wait()

                @pl.when(blk(it - _A_NBUF) == _NVB - 1)
                def _():
                    out_copy_last(it - _A_NBUF, b).wait()

            ibuf = ibufs[b]
            obuf = obufs[b]

            zero = iota * 0

            @plsc.parallel_loop(0, _VBLK // 2, unroll=4)
            def _row(k, ibuf=ibuf, obuf=obuf, zero=zero):
                for h in range(2):
                    vi = zero + (2 * k + h)
                    for kk in range(_D // _LANES):
                        di = iota + (kk * _LANES)
                        vals = plsc.load_gather(ibuf, [di, vi])
                        obuf[k, pl.ds(h * _D + kk * _LANES, _LANES)] = (
                            vals * _SCALE)

            @pl.when(bi < _NVB - 1)
            def _():
                out_copy(it, b).start()

            @pl.when(bi == _NVB - 1)
            def _():
                out_copy_last(it, b).start()

            @pl.when(blk(it + _A_NBUF) < _NVB)
            def _():
                in_copy(it + _A_NBUF, b).start()

    def group(g, carry):
        for b in range(_A_NBUF):
            step(g * _A_NBUF + b, b)
        return carry

    lax.fori_loop(0, -(-_A_ITERS // _A_NBUF), group, 0)

    for b in range(_A_NBUF):
        last_groups = -(-_A_ITERS // _A_NBUF)
        it = (last_groups - 1) * _A_NBUF + b
        for back in range(2):
            itb = it - back * _A_NBUF

            @pl.when((blk(itb) >= _NVB - 1 - (_NW - 1)) &
                     (blk(itb) < _NVB - 1))
            def _(itb=itb, b=b):
                out_copy(itb, b).wait()

            @pl.when(blk(itb) == _NVB - 1)
            def _(itb=itb, b=b):
                out_copy_last(itb, b).wait()


_prep_call = functools.partial(
    pl.kernel,
    out_type=jax.ShapeDtypeStruct((_VOCAB // 2, 2 * _D), jnp.float32),
    mesh=plsc.VectorSubcoreMesh(core_axis_name="c", subcore_axis_name="s"),
    compiler_params=pltpu.CompilerParams(needs_layout_passes=False),
    scratch_types=(
        [pltpu.VMEM((_D, _VBLK), jnp.float32) for _ in range(_A_NBUF)]
        + [pltpu.VMEM((_VBLK // 2, 2 * _D), jnp.float32)
           for _ in range(_A_NBUF)]
        + [pltpu.SemaphoreType.DMA for _ in range(2 * _A_NBUF)]
    ),
)(_prep_body)


def _lookup_body(tp_hbm, xt_hbm, out_hbm, idx_v, *rest):
    pidx = rest[0:_B_NBUF]
    lohi = rest[_B_NBUF:2 * _B_NBUF]
    gbufs = rest[2 * _B_NBUF:3 * _B_NBUF]
    obufs = rest[3 * _B_NBUF:4 * _B_NBUF]
    gsems = rest[4 * _B_NBUF:5 * _B_NBUF]
    osems = rest[5 * _B_NBUF:6 * _B_NBUF]

    wid = lax.axis_index("s") * _NC + lax.axis_index("c")
    col0 = wid * _BBLK
    iota = lax.iota(jnp.int32, _LANES)

    pltpu.sync_copy(xt_hbm.at[:, pl.ds(col0, _BBLK)], idx_v)

    def build_idx(c, b):
        for k in range(_BBLK // _LANES):
            sl = pl.ds(k * _LANES, _LANES)
            v = idx_v[c, sl]
            pidx[b][sl] = lax.shift_right_logical(v, 1)
            lohi[b][sl] = lax.bitwise_and(v, 1) * _D

    def gather_copy(b):
        return pltpu.make_async_copy(tp_hbm.at[pidx[b]], gbufs[b], gsems[b])

    def out_copy(c, b):
        return pltpu.make_async_copy(
            obufs[b], out_hbm.at[c, :, pl.ds(col0, _BBLK)], osems[b])

    for b in range(_B_NBUF):
        build_idx(b, b)
        gather_copy(b).start()

    def group(g, carry):
        for b in range(_B_NBUF):
            c = g * _B_NBUF + b
            gather_copy(b).wait()

            @pl.when(g > 0)
            def _():
                out_copy(c - _B_NBUF, b).wait()

            gbuf = gbufs[b]
            obuf = obufs[b]
            lob = lohi[b]

            jis = [iota + (k * _LANES) for k in range(_BBLK // _LANES)]
            his = [lob[pl.ds(k * _LANES, _LANES)]
                   for k in range(_BBLK // _LANES)]

            @plsc.parallel_loop(0, _D, unroll=4)
            def _drow(d, gbuf=gbuf, obuf=obuf):
                for k in range(_BBLK // _LANES):
                    vals = plsc.load_gather(gbuf, [jis[k], his[k] + d])
                    obuf[d, pl.ds(k * _LANES, _LANES)] = vals

            out_copy(c, b).start()

            @pl.when(g + 1 < _B_NGROUP)
            def _():
                build_idx(c + _B_NBUF, b)
                gather_copy(b).start()
        return carry

    lax.fori_loop(0, _B_NGROUP, group, 0)

    for b in range(_B_NBUF):
        out_copy((_B_NGROUP - 1) * _B_NBUF + b, b).wait()


_lookup_call = functools.partial(
    pl.kernel,
    out_type=jax.ShapeDtypeStruct((_L, _D, _B), jnp.float32),
    mesh=plsc.VectorSubcoreMesh(core_axis_name="c", subcore_axis_name="s"),
    compiler_params=pltpu.CompilerParams(needs_layout_passes=False),
    scratch_types=(
        [pltpu.VMEM((_L, _BBLK), jnp.int32)]
        + [pltpu.VMEM((_BBLK,), jnp.int32) for _ in range(2 * _B_NBUF)]
        + [pltpu.VMEM((_BBLK, 2 * _D), jnp.float32) for _ in range(_B_NBUF)]
        + [pltpu.VMEM((_D, _BBLK), jnp.float32) for _ in range(_B_NBUF)]
        + [pltpu.SemaphoreType.DMA for _ in range(2 * _B_NBUF)]
    ),
)(_lookup_body)


@jax.jit
def kernel(embedding_table, x):
    tp = _prep_call(embedding_table.T)
    out = _lookup_call(tp, x.T)
    return out.transpose(2, 0, 1)
